# baseline (speedup 1.0000x reference)
import jax
import jax.numpy as jnp
from jax.experimental import pallas as pl

N_F = 128
N_ROWS = 16 * 64 * 64
CHUNK = 8192
N_STEPS = N_ROWS // CHUNK


def _stats_kernel(x_ref, sum_ref, xx_ref, cnt_ref, min_ref, max_ref):
    i = pl.program_id(0)
    x = x_ref[...]

    mask = jnp.isnan(x)
    nn = jnp.where(mask, 0.0, 1.0)
    pcnt = jnp.sum(nn, axis=0)[None, :]
    clean = jnp.sum(pcnt) == float(CHUNK * N_F)

    @pl.when(i == 0)
    def _init_cnt():
        cnt_ref[...] = pcnt

    @pl.when(i != 0)
    def _acc_cnt():
        cnt_ref[...] += pcnt

    def _combine(psum, pmin, pmax, pxx, first):
        if first:
            sum_ref[...] = psum
            min_ref[...] = pmin
            max_ref[...] = pmax
            xx_ref[...] = pxx
        else:
            sum_ref[...] += psum
            min_ref[...] = jnp.minimum(min_ref[...], pmin)
            max_ref[...] = jnp.maximum(max_ref[...], pmax)
            xx_ref[...] += pxx

    def _fast():
        psum = jnp.sum(x, axis=0)[None, :]
        pmin = jnp.min(x, axis=0)[None, :]
        pmax = jnp.max(x, axis=0)[None, :]
        pxx = jax.lax.dot_general(
            x, x, (((0,), (0,)), ((), ())), preferred_element_type=jnp.float32
        )
        return psum, pmin, pmax, pxx

    def _masked():
        xm = jnp.where(mask, 0.0, x)
        psum = jnp.sum(xm, axis=0)[None, :]
        pmin = jnp.min(jnp.where(mask, jnp.inf, x), axis=0)[None, :]
        pmax = jnp.max(jnp.where(mask, -jnp.inf, x), axis=0)[None, :]
        pxx = jax.lax.dot_general(
            xm, xm, (((0,), (0,)), ((), ())), preferred_element_type=jnp.float32
        )
        return psum, pmin, pmax, pxx

    for first in (True, False):
        first_pred = (i == 0) if first else (i != 0)

        @pl.when(jnp.logical_and(clean, first_pred))
        def _():
            _combine(*_fast(), first)

        @pl.when(jnp.logical_and(jnp.logical_not(clean), first_pred))
        def _():
            _combine(*_masked(), first)


def kernel(x):
    xt = jnp.transpose(x, (0, 2, 3, 1)).reshape(N_ROWS, N_F)
    vec = jax.ShapeDtypeStruct((1, N_F), jnp.float32)
    out = pl.pallas_call(
        _stats_kernel,
        grid=(N_STEPS,),
        in_specs=[pl.BlockSpec((CHUNK, N_F), lambda i: (i, 0))],
        out_specs=[
            pl.BlockSpec((1, N_F), lambda i: (0, 0)),
            pl.BlockSpec((N_F, N_F), lambda i: (0, 0)),
            pl.BlockSpec((1, N_F), lambda i: (0, 0)),
            pl.BlockSpec((1, N_F), lambda i: (0, 0)),
            pl.BlockSpec((1, N_F), lambda i: (0, 0)),
        ],
        out_shape=[
            vec,
            jax.ShapeDtypeStruct((N_F, N_F), jnp.float32),
            vec,
            vec,
            vec,
        ],
    )(xt)
    x_sum, xx_sum, counts, min_vals, max_vals = out
    return (
        x_sum.reshape(N_F),
        xx_sum,
        counts.reshape(N_F),
        min_vals.reshape(N_F),
        max_vals.reshape(N_F),
    )


# speedup vs baseline: 1.1580x; 1.1580x over previous
import jax
import jax.numpy as jnp
from jax.experimental import pallas as pl

N_F = 128
N_ROWS = 16 * 64 * 64
CHUNK = 8192
N_STEPS = N_ROWS // CHUNK


def _stats_kernel(x_ref, sum_ref, xx_ref, cnt_ref, min_ref, max_ref):
    i = pl.program_id(0)
    x = x_ref[...]
    mask = jnp.isnan(x)
    xm = jnp.where(mask, 0.0, x)

    psum = jnp.sum(xm, axis=0)[None, :]
    pcnt = jnp.sum(jnp.where(mask, 0.0, 1.0), axis=0)[None, :]
    pmin = jnp.min(jnp.where(mask, jnp.inf, x), axis=0)[None, :]
    pmax = jnp.max(jnp.where(mask, -jnp.inf, x), axis=0)[None, :]
    pxx = jax.lax.dot_general(
        xm, xm, (((0,), (0,)), ((), ())), preferred_element_type=jnp.float32
    )

    @pl.when(i == 0)
    def _init():
        sum_ref[...] = psum
        cnt_ref[...] = pcnt
        min_ref[...] = pmin
        max_ref[...] = pmax
        xx_ref[...] = pxx

    @pl.when(i != 0)
    def _acc():
        sum_ref[...] += psum
        cnt_ref[...] += pcnt
        min_ref[...] = jnp.minimum(min_ref[...], pmin)
        max_ref[...] = jnp.maximum(max_ref[...], pmax)
        xx_ref[...] += pxx


def kernel(x):
    xt = jnp.transpose(x, (0, 2, 3, 1)).reshape(N_ROWS, N_F)
    vec = jax.ShapeDtypeStruct((1, N_F), jnp.float32)
    out = pl.pallas_call(
        _stats_kernel,
        grid=(N_STEPS,),
        in_specs=[pl.BlockSpec((CHUNK, N_F), lambda i: (i, 0))],
        out_specs=[
            pl.BlockSpec((1, N_F), lambda i: (0, 0)),
            pl.BlockSpec((N_F, N_F), lambda i: (0, 0)),
            pl.BlockSpec((1, N_F), lambda i: (0, 0)),
            pl.BlockSpec((1, N_F), lambda i: (0, 0)),
            pl.BlockSpec((1, N_F), lambda i: (0, 0)),
        ],
        out_shape=[
            vec,
            jax.ShapeDtypeStruct((N_F, N_F), jnp.float32),
            vec,
            vec,
            vec,
        ],
    )(xt)
    x_sum, xx_sum, counts, min_vals, max_vals = out
    return (
        x_sum.reshape(N_F),
        xx_sum,
        counts.reshape(N_F),
        min_vals.reshape(N_F),
        max_vals.reshape(N_F),
    )
